# baseline (speedup 1.0000x reference)
import dataclasses
import functools

import jax
import jax.numpy as jnp
from jax import lax
from jax.experimental import pallas as pl
from jax.experimental.pallas import tpu as pltpu
from jax.experimental.pallas import tpu_sc as plsc

NC = 2
NS = 16
NW = NC * NS
CH = 128
GRP = 8
G = 64

_HI = jax.lax.Precision.HIGHEST

_SC_PARAMS = pltpu.CompilerParams()
if "needs_layout_passes" in pltpu.CompilerParams.__dataclass_fields__:
    _SC_PARAMS = dataclasses.replace(_SC_PARAMS, needs_layout_passes=False)


def _pad_rows(n):
    q = NS * 8
    return ((n + q - 1) // q) * q


@functools.partial(jax.jit, static_argnames=("n", "e"))
def _sc_degree(dst, *, n, e):
    npad = _pad_rows(n)
    ept = e // NW

    @functools.partial(
        pl.kernel,
        out_type=jax.ShapeDtypeStruct((NW, npad), jnp.float32),
        mesh=plsc.VectorSubcoreMesh(core_axis_name="c", subcore_axis_name="s"),
        scratch_types=[
            pltpu.VMEM((ept,), jnp.int32),
            pltpu.VMEM((npad,), jnp.float32),
            pltpu.SemaphoreType.DMA,
        ],
        compiler_params=_SC_PARAMS,
    )
    def deg_kernel(dst_hbm, out_hbm, idx_v, hist, sem):
        cid = lax.axis_index("c")
        sid = lax.axis_index("s")
        wid = sid * NC + cid
        pltpu.sync_copy(dst_hbm.at[pl.ds(wid * ept, ept)], idx_v)
        zeros16 = jnp.zeros((16,), jnp.float32)

        @pl.loop(0, npad // 16)
        def _(i):
            hist[pl.ds(i * 16, 16)] = zeros16

        ones16 = jnp.ones((16,), jnp.float32)

        @pl.loop(0, ept // 16)
        def _(j):
            idx = idx_v[pl.ds(j * 16, 16)]
            plsc.addupdate_scatter(hist, [idx], ones16)

        pltpu.sync_copy(hist, out_hbm.at[wid])

    return deg_kernel(dst)


@functools.partial(jax.jit, static_argnames=("n", "nch", "h"))
def _sc_edge_agg(zp, src3, dst3, zeros_hbm, *, n, nch, h):
    npad = _pad_rows(n)
    ngrp = nch // GRP
    assert ngrp % 2 == 0
    rpt = npad // NS

    @functools.partial(
        pl.kernel,
        out_type=jax.ShapeDtypeStruct((NC, npad, h), jnp.float32),
        mesh=plsc.VectorSubcoreMesh(core_axis_name="c", subcore_axis_name="s"),
        scratch_types=[
            pltpu.VMEM((GRP, CH), jnp.int32),
            pltpu.VMEM((GRP, CH), jnp.int32),
            pltpu.VMEM((CH, h), jnp.float32),
            pltpu.VMEM_SHARED((npad, h), jnp.float32),
            pltpu.SemaphoreType.DMA,
        ],
    )
    def agg_kernel(zp_hbm, src_hbm, dst_hbm, zeros_h, out_hbm,
                   srcg, dstg, rows, acc, gsem):
        cid = lax.axis_index("c")
        sid = lax.axis_index("s")
        wid = sid * NC + cid
        r0 = sid * rpt
        pltpu.sync_copy(zeros_h.at[pl.ds(r0, rpt)], acc.at[pl.ds(r0, rpt)])
        plsc.subcore_barrier()

        @pl.loop(0, ngrp)
        def _(gg):
            sl = pl.ds(gg * GRP, GRP)
            pltpu.sync_copy(src_hbm.at[wid, sl], srcg)
            pltpu.sync_copy(dst_hbm.at[wid, sl], dstg)

            @pl.loop(0, GRP)
            def _(c):
                pltpu.async_copy(zp_hbm.at[srcg.at[c]], rows, gsem).wait()
                pltpu.sync_copy(rows, acc.at[dstg.at[c]], add=True)

        plsc.subcore_barrier()
        pltpu.sync_copy(acc.at[pl.ds(r0, rpt)], out_hbm.at[cid, pl.ds(r0, rpt)])

    return agg_kernel(zp, src3, dst3, zeros_hbm)


def _tc_matmul(x, w):
    def body(x_ref, w_ref, z_ref):
        z_ref[...] = jnp.dot(x_ref[...], w_ref[...], precision=_HI,
                             preferred_element_type=jnp.float32)

    return pl.pallas_call(
        body,
        out_shape=jax.ShapeDtypeStruct((x.shape[0], w.shape[1]), jnp.float32),
    )(x, w)


def _tc_prep(degp, ones_nw, z):
    n, h = z.shape

    npad = _pad_rows(n)

    def body(degp_ref, ones_ref, z_ref, dinv_ref, zp_ref):
        degc = jax.lax.dot_general(
            degp_ref[...], ones_ref[...],
            dimension_numbers=(((0,), (0,)), ((), ())),
            precision=_HI, preferred_element_type=jnp.float32)
        dinv = jax.lax.rsqrt(degc[:n] + 1.0)
        dinv_ref[...] = dinv
        zp_ref[...] = jnp.concatenate(
            [z_ref[...] * dinv, jnp.zeros((npad - n, h), jnp.float32)], axis=0)

    return pl.pallas_call(
        body,
        out_shape=(jax.ShapeDtypeStruct((n, 1), jnp.float32),
                   jax.ShapeDtypeStruct((npad, h), jnp.float32)),
    )(degp, ones_nw, z)


def _combine_bn_relu(p_ref, z_ref, dinv_ref, g_ref, be_ref):
    n = z_ref.shape[0]
    dinv = dinv_ref[...]
    pre = dinv * (p_ref[0, :n] + p_ref[1, :n]) + (dinv * dinv) * z_ref[...]
    m = jnp.mean(pre, axis=0, keepdims=True)
    v = jnp.mean((pre - m) ** 2, axis=0, keepdims=True)
    hh = g_ref[...] * (pre - m) * jax.lax.rsqrt(v + 1e-5) + be_ref[...]
    return jnp.maximum(hh, 0.0)


def _tc_mid(p, z, dinv, g, be, w_next):
    n, h = z.shape
    npad = _pad_rows(n)
    hn = w_next.shape[1]

    def body(p_ref, z_ref, dinv_ref, g_ref, be_ref, w_ref, zn_ref, zpn_ref):
        hh = _combine_bn_relu(p_ref, z_ref, dinv_ref, g_ref, be_ref)
        zn = jnp.dot(hh, w_ref[...], precision=_HI,
                     preferred_element_type=jnp.float32)
        zn_ref[...] = zn
        zpn_ref[...] = jnp.concatenate(
            [zn * dinv_ref[...], jnp.zeros((npad - n, hn), jnp.float32)],
            axis=0)

    return pl.pallas_call(
        body,
        out_shape=(jax.ShapeDtypeStruct((n, hn), jnp.float32),
                   jax.ShapeDtypeStruct((npad, hn), jnp.float32)),
    )(p, z, dinv, g, be, w_next)


def _tc_final(p, z, dinv, g, be, batch2d, wl, bl):
    n, h = z.shape

    def body(p_ref, z_ref, dinv_ref, g_ref, be_ref, b_ref, wl_ref, bl_ref,
             out_ref):
        hh = _combine_bn_relu(p_ref, z_ref, dinv_ref, g_ref, be_ref)
        gids = jax.lax.broadcasted_iota(jnp.int32, (G, n), 0)
        onehot = jnp.where(gids == b_ref[...], 1.0, 0.0)
        sums = jnp.dot(onehot, hh, precision=_HI,
                       preferred_element_type=jnp.float32)
        counts = jnp.sum(onehot, axis=1, keepdims=True)
        pooled = sums / jnp.maximum(counts, 1.0)
        out_ref[...] = jnp.dot(pooled, wl_ref[...], precision=_HI,
                               preferred_element_type=jnp.float32) + bl_ref[...]

    return pl.pallas_call(
        body,
        out_shape=jax.ShapeDtypeStruct((G, 1), jnp.float32),
    )(p, z, dinv, g, be, batch2d, wl, bl)


def kernel(x, edge_index, batch, W1, b1, g1, be1, W2, b2, g2, be2,
           W3, b3, g3, be3, Wl, bl):
    n, f_in = x.shape
    e = edge_index.shape[1]
    h = W1.shape[1]
    npad = _pad_rows(n)

    assert e % NW == 0
    ept = e // NW
    grain = GRP * CH * 2
    eptp = ((ept + grain - 1) // grain) * grain
    nch = eptp // CH
    dpt = eptp - ept
    assert dpt == 0 or npad > n
    idt = edge_index.dtype
    dummy_src = jnp.full((NW, dpt), n, dtype=idt)
    dummy_dst = jnp.arange(NW * dpt, dtype=idt).reshape(NW, dpt) % npad
    src3 = jnp.concatenate(
        [edge_index[0].reshape(NW, ept), dummy_src], axis=1).reshape(NW, nch, CH)
    dst3 = jnp.concatenate(
        [edge_index[1].reshape(NW, ept), dummy_dst], axis=1).reshape(NW, nch, CH)
    zeros_h = jnp.zeros((npad, h), jnp.float32)
    ones_nw = jnp.ones((NW, 1), jnp.float32)
    batch2d = batch.reshape(1, n)
    g1r, be1r = g1.reshape(1, h), be1.reshape(1, h)
    g2r, be2r = g2.reshape(1, h), be2.reshape(1, h)
    g3r, be3r = g3.reshape(1, h), be3.reshape(1, h)
    blr = bl.reshape(1, 1)

    degp = _sc_degree(edge_index[1], n=n, e=e)
    z1 = _tc_matmul(x, W1)
    dinv, zp1 = _tc_prep(degp, ones_nw, z1)
    p1 = _sc_edge_agg(zp1, src3, dst3, zeros_h, n=n, nch=nch, h=h)
    z2, zp2 = _tc_mid(p1, z1, dinv, g1r, be1r, W2)
    p2 = _sc_edge_agg(zp2, src3, dst3, zeros_h, n=n, nch=nch, h=h)
    z3, zp3 = _tc_mid(p2, z2, dinv, g2r, be2r, W3)
    p3 = _sc_edge_agg(zp3, src3, dst3, zeros_h, n=n, nch=nch, h=h)
    return _tc_final(p3, z3, dinv, g3r, be3r, batch2d, Wl, blr)


# speedup vs baseline: 1.0160x; 1.0160x over previous
import dataclasses
import functools

import jax
import jax.numpy as jnp
from jax import lax
from jax.experimental import pallas as pl
from jax.experimental.pallas import tpu as pltpu
from jax.experimental.pallas import tpu_sc as plsc

NC = 2
NS = 16
NW = NC * NS
CH = 128
GRP = 8
G = 64

_HI = jax.lax.Precision.HIGHEST

_SC_PARAMS = pltpu.CompilerParams()
if "needs_layout_passes" in pltpu.CompilerParams.__dataclass_fields__:
    _SC_PARAMS = dataclasses.replace(_SC_PARAMS, needs_layout_passes=False)


def _pad_rows(n):
    q = NS * 8
    return ((n + q - 1) // q) * q


@functools.partial(jax.jit, static_argnames=("n", "e"))
def _sc_degree(dst, *, n, e):
    npad = _pad_rows(n)
    ept = e // NW

    @functools.partial(
        pl.kernel,
        out_type=jax.ShapeDtypeStruct((NW, npad), jnp.float32),
        mesh=plsc.VectorSubcoreMesh(core_axis_name="c", subcore_axis_name="s"),
        scratch_types=[
            pltpu.VMEM((ept,), jnp.int32),
            pltpu.VMEM((npad,), jnp.float32),
            pltpu.SemaphoreType.DMA,
        ],
        compiler_params=_SC_PARAMS,
    )
    def deg_kernel(dst_hbm, out_hbm, idx_v, hist, sem):
        cid = lax.axis_index("c")
        sid = lax.axis_index("s")
        wid = sid * NC + cid
        pltpu.sync_copy(dst_hbm.at[pl.ds(wid * ept, ept)], idx_v)
        zeros16 = jnp.zeros((16,), jnp.float32)

        @pl.loop(0, npad // 16)
        def _(i):
            hist[pl.ds(i * 16, 16)] = zeros16

        ones16 = jnp.ones((16,), jnp.float32)

        @pl.loop(0, ept // 16)
        def _(j):
            idx = idx_v[pl.ds(j * 16, 16)]
            plsc.addupdate_scatter(hist, [idx], ones16)

        pltpu.sync_copy(hist, out_hbm.at[wid])

    return deg_kernel(dst)


@functools.partial(jax.jit, static_argnames=("n", "nch", "h"))
def _sc_edge_agg(zp, src3, dst3, zeros_hbm, *, n, nch, h):
    npad = _pad_rows(n)
    ngrp = nch // GRP
    assert ngrp % 2 == 0
    rpt = npad // NS

    @functools.partial(
        pl.kernel,
        out_type=jax.ShapeDtypeStruct((NC, npad, h), jnp.float32),
        mesh=plsc.VectorSubcoreMesh(core_axis_name="c", subcore_axis_name="s"),
        scratch_types=[
            pltpu.VMEM((nch, CH), jnp.int32),
            pltpu.VMEM((nch, CH), jnp.int32),
            pltpu.VMEM((CH, h), jnp.float32),
            pltpu.VMEM_SHARED((npad, h), jnp.float32),
            pltpu.SemaphoreType.DMA,
        ],
    )
    def agg_kernel(zp_hbm, src_hbm, dst_hbm, zeros_h, out_hbm,
                   srcv, dstv, rows, acc, gsem):
        cid = lax.axis_index("c")
        sid = lax.axis_index("s")
        wid = sid * NC + cid
        r0 = sid * rpt
        pltpu.sync_copy(src_hbm.at[wid], srcv)
        pltpu.sync_copy(dst_hbm.at[wid], dstv)
        pltpu.sync_copy(zeros_h.at[pl.ds(r0, rpt)], acc.at[pl.ds(r0, rpt)])
        plsc.subcore_barrier()

        @pl.loop(0, nch)
        def _(c):
            pltpu.async_copy(zp_hbm.at[srcv.at[c]], rows, gsem).wait()
            pltpu.sync_copy(rows, acc.at[dstv.at[c]], add=True)

        plsc.subcore_barrier()
        pltpu.sync_copy(acc.at[pl.ds(r0, rpt)], out_hbm.at[cid, pl.ds(r0, rpt)])

    return agg_kernel(zp, src3, dst3, zeros_hbm)


def _tc_matmul(x, w):
    def body(x_ref, w_ref, z_ref):
        z_ref[...] = jnp.dot(x_ref[...], w_ref[...], precision=_HI,
                             preferred_element_type=jnp.float32)

    return pl.pallas_call(
        body,
        out_shape=jax.ShapeDtypeStruct((x.shape[0], w.shape[1]), jnp.float32),
    )(x, w)


def _tc_prep(degp, ones_nw, z):
    n, h = z.shape

    npad = _pad_rows(n)

    def body(degp_ref, ones_ref, z_ref, dinv_ref, zp_ref):
        degc = jax.lax.dot_general(
            degp_ref[...], ones_ref[...],
            dimension_numbers=(((0,), (0,)), ((), ())),
            precision=_HI, preferred_element_type=jnp.float32)
        dinv = jax.lax.rsqrt(degc[:n] + 1.0)
        dinv_ref[...] = dinv
        zp_ref[...] = jnp.concatenate(
            [z_ref[...] * dinv, jnp.zeros((npad - n, h), jnp.float32)], axis=0)

    return pl.pallas_call(
        body,
        out_shape=(jax.ShapeDtypeStruct((n, 1), jnp.float32),
                   jax.ShapeDtypeStruct((npad, h), jnp.float32)),
    )(degp, ones_nw, z)


def _combine_bn_relu(p_ref, z_ref, dinv_ref, g_ref, be_ref):
    n = z_ref.shape[0]
    dinv = dinv_ref[...]
    pre = dinv * (p_ref[0, :n] + p_ref[1, :n]) + (dinv * dinv) * z_ref[...]
    m = jnp.mean(pre, axis=0, keepdims=True)
    v = jnp.mean((pre - m) ** 2, axis=0, keepdims=True)
    hh = g_ref[...] * (pre - m) * jax.lax.rsqrt(v + 1e-5) + be_ref[...]
    return jnp.maximum(hh, 0.0)


def _tc_mid(p, z, dinv, g, be, w_next):
    n, h = z.shape
    npad = _pad_rows(n)
    hn = w_next.shape[1]

    def body(p_ref, z_ref, dinv_ref, g_ref, be_ref, w_ref, zn_ref, zpn_ref):
        hh = _combine_bn_relu(p_ref, z_ref, dinv_ref, g_ref, be_ref)
        zn = jnp.dot(hh, w_ref[...], precision=_HI,
                     preferred_element_type=jnp.float32)
        zn_ref[...] = zn
        zpn_ref[...] = jnp.concatenate(
            [zn * dinv_ref[...], jnp.zeros((npad - n, hn), jnp.float32)],
            axis=0)

    return pl.pallas_call(
        body,
        out_shape=(jax.ShapeDtypeStruct((n, hn), jnp.float32),
                   jax.ShapeDtypeStruct((npad, hn), jnp.float32)),
    )(p, z, dinv, g, be, w_next)


def _tc_final(p, z, dinv, g, be, batch2d, wl, bl):
    n, h = z.shape

    def body(p_ref, z_ref, dinv_ref, g_ref, be_ref, b_ref, wl_ref, bl_ref,
             out_ref):
        hh = _combine_bn_relu(p_ref, z_ref, dinv_ref, g_ref, be_ref)
        gids = jax.lax.broadcasted_iota(jnp.int32, (G, n), 0)
        onehot = jnp.where(gids == b_ref[...], 1.0, 0.0)
        sums = jnp.dot(onehot, hh, precision=_HI,
                       preferred_element_type=jnp.float32)
        counts = jnp.sum(onehot, axis=1, keepdims=True)
        pooled = sums / jnp.maximum(counts, 1.0)
        out_ref[...] = jnp.dot(pooled, wl_ref[...], precision=_HI,
                               preferred_element_type=jnp.float32) + bl_ref[...]

    return pl.pallas_call(
        body,
        out_shape=jax.ShapeDtypeStruct((G, 1), jnp.float32),
    )(p, z, dinv, g, be, batch2d, wl, bl)


def kernel(x, edge_index, batch, W1, b1, g1, be1, W2, b2, g2, be2,
           W3, b3, g3, be3, Wl, bl):
    n, f_in = x.shape
    e = edge_index.shape[1]
    h = W1.shape[1]
    npad = _pad_rows(n)

    assert e % NW == 0
    ept = e // NW
    grain = GRP * CH * 2
    eptp = ((ept + grain - 1) // grain) * grain
    nch = eptp // CH
    dpt = eptp - ept
    assert dpt == 0 or npad > n
    idt = edge_index.dtype
    dummy_src = jnp.full((NW, dpt), n, dtype=idt)
    dummy_dst = jnp.arange(NW * dpt, dtype=idt).reshape(NW, dpt) % npad
    src3 = jnp.concatenate(
        [edge_index[0].reshape(NW, ept), dummy_src], axis=1).reshape(NW, nch, CH)
    dst3 = jnp.concatenate(
        [edge_index[1].reshape(NW, ept), dummy_dst], axis=1).reshape(NW, nch, CH)
    zeros_h = jnp.zeros((npad, h), jnp.float32)
    ones_nw = jnp.ones((NW, 1), jnp.float32)
    batch2d = batch.reshape(1, n)
    g1r, be1r = g1.reshape(1, h), be1.reshape(1, h)
    g2r, be2r = g2.reshape(1, h), be2.reshape(1, h)
    g3r, be3r = g3.reshape(1, h), be3.reshape(1, h)
    blr = bl.reshape(1, 1)

    degp = _sc_degree(edge_index[1], n=n, e=e)
    z1 = _tc_matmul(x, W1)
    dinv, zp1 = _tc_prep(degp, ones_nw, z1)
    p1 = _sc_edge_agg(zp1, src3, dst3, zeros_h, n=n, nch=nch, h=h)
    z2, zp2 = _tc_mid(p1, z1, dinv, g1r, be1r, W2)
    p2 = _sc_edge_agg(zp2, src3, dst3, zeros_h, n=n, nch=nch, h=h)
    z3, zp3 = _tc_mid(p2, z2, dinv, g2r, be2r, W3)
    p3 = _sc_edge_agg(zp3, src3, dst3, zeros_h, n=n, nch=nch, h=h)
    return _tc_final(p3, z3, dinv, g3r, be3r, batch2d, Wl, blr)


# speedup vs baseline: 2.0523x; 2.0199x over previous
import dataclasses
import functools

import jax
import jax.numpy as jnp
from jax import lax
from jax.experimental import pallas as pl
from jax.experimental.pallas import tpu as pltpu
from jax.experimental.pallas import tpu_sc as plsc

NC = 2
NS = 16
NW = NC * NS
CH = 80
GRP = 1
G = 64

_HI = jax.lax.Precision.HIGHEST

_SC_PARAMS = pltpu.CompilerParams()
if "needs_layout_passes" in pltpu.CompilerParams.__dataclass_fields__:
    _SC_PARAMS = dataclasses.replace(_SC_PARAMS, needs_layout_passes=False)


def _pad_rows(n):
    q = NS * 8
    return ((n + q - 1) // q) * q


@functools.partial(jax.jit, static_argnames=("n", "e"))
def _sc_degree(dst, *, n, e):
    npad = _pad_rows(n)
    ept = e // NW

    @functools.partial(
        pl.kernel,
        out_type=jax.ShapeDtypeStruct((NW, npad), jnp.float32),
        mesh=plsc.VectorSubcoreMesh(core_axis_name="c", subcore_axis_name="s"),
        scratch_types=[
            pltpu.VMEM((ept,), jnp.int32),
            pltpu.VMEM((npad,), jnp.float32),
            pltpu.SemaphoreType.DMA,
        ],
        compiler_params=_SC_PARAMS,
    )
    def deg_kernel(dst_hbm, out_hbm, idx_v, hist, sem):
        cid = lax.axis_index("c")
        sid = lax.axis_index("s")
        wid = sid * NC + cid
        pltpu.sync_copy(dst_hbm.at[pl.ds(wid * ept, ept)], idx_v)
        zeros16 = jnp.zeros((16,), jnp.float32)

        @pl.loop(0, npad // 16)
        def _(i):
            hist[pl.ds(i * 16, 16)] = zeros16

        ones16 = jnp.ones((16,), jnp.float32)

        @pl.loop(0, ept // 16)
        def _(j):
            idx = idx_v[pl.ds(j * 16, 16)]
            plsc.addupdate_scatter(hist, [idx], ones16)

        pltpu.sync_copy(hist, out_hbm.at[wid])

    return deg_kernel(dst)


@functools.partial(jax.jit, static_argnames=("n", "nch", "h"))
def _sc_edge_agg(zp, src3, dst3, zeros_hbm, *, n, nch, h):
    npad = _pad_rows(n)
    rpt = npad // NS

    @functools.partial(
        pl.kernel,
        out_type=jax.ShapeDtypeStruct((NC, npad, h), jnp.float32),
        mesh=plsc.VectorSubcoreMesh(core_axis_name="c", subcore_axis_name="s"),
        scratch_types=[
            pltpu.VMEM((nch, CH), jnp.int32),
            pltpu.VMEM((nch, CH), jnp.int32),
            pltpu.VMEM((CH, h), jnp.float32),
            pltpu.VMEM_SHARED((npad, h), jnp.float32),
            pltpu.SemaphoreType.DMA,
        ],
    )
    def agg_kernel(zp_hbm, src_hbm, dst_hbm, zeros_h, out_hbm,
                   srcv, dstv, rows, acc, gsem):
        cid = lax.axis_index("c")
        sid = lax.axis_index("s")
        wid = sid * NC + cid
        r0 = sid * rpt
        pltpu.sync_copy(src_hbm.at[wid], srcv)
        pltpu.sync_copy(dst_hbm.at[wid], dstv)
        pltpu.sync_copy(zeros_h.at[pl.ds(r0, rpt)], acc.at[pl.ds(r0, rpt)])
        plsc.subcore_barrier()

        @pl.loop(0, nch)
        def _(c):
            pltpu.async_copy(zp_hbm.at[srcv.at[c]], rows, gsem).wait()
            pltpu.sync_copy(rows, acc.at[dstv.at[c]], add=True)

        plsc.subcore_barrier()
        pltpu.sync_copy(acc.at[pl.ds(r0, rpt)], out_hbm.at[cid, pl.ds(r0, rpt)])

    return agg_kernel(zp, src3, dst3, zeros_hbm)


def _tc_matmul(x, w):
    def body(x_ref, w_ref, z_ref):
        z_ref[...] = jnp.dot(x_ref[...], w_ref[...], precision=_HI,
                             preferred_element_type=jnp.float32)

    return pl.pallas_call(
        body,
        out_shape=jax.ShapeDtypeStruct((x.shape[0], w.shape[1]), jnp.float32),
    )(x, w)


def _tc_prep(degp, ones_nw, z):
    n, h = z.shape

    npad = _pad_rows(n)

    def body(degp_ref, ones_ref, z_ref, dinv_ref, zp_ref):
        degc = jax.lax.dot_general(
            degp_ref[...], ones_ref[...],
            dimension_numbers=(((0,), (0,)), ((), ())),
            precision=_HI, preferred_element_type=jnp.float32)
        dinv = jax.lax.rsqrt(degc[:n] + 1.0)
        dinv_ref[...] = dinv
        zp_ref[...] = jnp.concatenate(
            [z_ref[...] * dinv, jnp.zeros((npad - n, h), jnp.float32)], axis=0)

    return pl.pallas_call(
        body,
        out_shape=(jax.ShapeDtypeStruct((n, 1), jnp.float32),
                   jax.ShapeDtypeStruct((npad, h), jnp.float32)),
    )(degp, ones_nw, z)


def _combine_bn_relu(p_ref, z_ref, dinv_ref, g_ref, be_ref):
    n = z_ref.shape[0]
    dinv = dinv_ref[...]
    pre = dinv * (p_ref[0, :n] + p_ref[1, :n]) + (dinv * dinv) * z_ref[...]
    m = jnp.mean(pre, axis=0, keepdims=True)
    v = jnp.mean((pre - m) ** 2, axis=0, keepdims=True)
    hh = g_ref[...] * (pre - m) * jax.lax.rsqrt(v + 1e-5) + be_ref[...]
    return jnp.maximum(hh, 0.0)


def _tc_mid(p, z, dinv, g, be, w_next):
    n, h = z.shape
    npad = _pad_rows(n)
    hn = w_next.shape[1]

    def body(p_ref, z_ref, dinv_ref, g_ref, be_ref, w_ref, zn_ref, zpn_ref):
        hh = _combine_bn_relu(p_ref, z_ref, dinv_ref, g_ref, be_ref)
        zn = jnp.dot(hh, w_ref[...], precision=_HI,
                     preferred_element_type=jnp.float32)
        zn_ref[...] = zn
        zpn_ref[...] = jnp.concatenate(
            [zn * dinv_ref[...], jnp.zeros((npad - n, hn), jnp.float32)],
            axis=0)

    return pl.pallas_call(
        body,
        out_shape=(jax.ShapeDtypeStruct((n, hn), jnp.float32),
                   jax.ShapeDtypeStruct((npad, hn), jnp.float32)),
    )(p, z, dinv, g, be, w_next)


def _tc_final(p, z, dinv, g, be, batch2d, wl, bl):
    n, h = z.shape

    def body(p_ref, z_ref, dinv_ref, g_ref, be_ref, b_ref, wl_ref, bl_ref,
             out_ref):
        hh = _combine_bn_relu(p_ref, z_ref, dinv_ref, g_ref, be_ref)
        gids = jax.lax.broadcasted_iota(jnp.int32, (G, n), 0)
        onehot = jnp.where(gids == b_ref[...], 1.0, 0.0)
        sums = jnp.dot(onehot, hh, precision=_HI,
                       preferred_element_type=jnp.float32)
        counts = jnp.sum(onehot, axis=1, keepdims=True)
        pooled = sums / jnp.maximum(counts, 1.0)
        out_ref[...] = jnp.dot(pooled, wl_ref[...], precision=_HI,
                               preferred_element_type=jnp.float32) + bl_ref[...]

    return pl.pallas_call(
        body,
        out_shape=jax.ShapeDtypeStruct((G, 1), jnp.float32),
    )(p, z, dinv, g, be, batch2d, wl, bl)


def kernel(x, edge_index, batch, W1, b1, g1, be1, W2, b2, g2, be2,
           W3, b3, g3, be3, Wl, bl):
    n, f_in = x.shape
    e = edge_index.shape[1]
    h = W1.shape[1]
    npad = _pad_rows(n)

    assert e % NW == 0
    ept = e // NW
    grain = GRP * CH
    eptp = ((ept + grain - 1) // grain) * grain
    nch = eptp // CH
    dpt = eptp - ept
    assert dpt == 0 or npad > n
    idt = edge_index.dtype
    dummy_src = jnp.full((NW, dpt), n, dtype=idt)
    dummy_dst = jnp.arange(NW * dpt, dtype=idt).reshape(NW, dpt) % npad
    src3 = jnp.concatenate(
        [edge_index[0].reshape(NW, ept), dummy_src], axis=1).reshape(NW, nch, CH)
    dst3 = jnp.concatenate(
        [edge_index[1].reshape(NW, ept), dummy_dst], axis=1).reshape(NW, nch, CH)
    zeros_h = jnp.zeros((npad, h), jnp.float32)
    ones_nw = jnp.ones((NW, 1), jnp.float32)
    batch2d = batch.reshape(1, n)
    g1r, be1r = g1.reshape(1, h), be1.reshape(1, h)
    g2r, be2r = g2.reshape(1, h), be2.reshape(1, h)
    g3r, be3r = g3.reshape(1, h), be3.reshape(1, h)
    blr = bl.reshape(1, 1)

    degp = _sc_degree(edge_index[1], n=n, e=e)
    z1 = _tc_matmul(x, W1)
    dinv, zp1 = _tc_prep(degp, ones_nw, z1)
    p1 = _sc_edge_agg(zp1, src3, dst3, zeros_h, n=n, nch=nch, h=h)
    z2, zp2 = _tc_mid(p1, z1, dinv, g1r, be1r, W2)
    p2 = _sc_edge_agg(zp2, src3, dst3, zeros_h, n=n, nch=nch, h=h)
    z3, zp3 = _tc_mid(p2, z2, dinv, g2r, be2r, W3)
    p3 = _sc_edge_agg(zp3, src3, dst3, zeros_h, n=n, nch=nch, h=h)
    return _tc_final(p3, z3, dinv, g3r, be3r, batch2d, Wl, blr)


# speedup vs baseline: 3.1913x; 1.5550x over previous
import dataclasses
import functools

import jax
import jax.numpy as jnp
from jax import lax
from jax.experimental import pallas as pl
from jax.experimental.pallas import tpu as pltpu
from jax.experimental.pallas import tpu_sc as plsc

NC = 2
NS = 16
NW = NC * NS
CH = 80
GRP = 1
G = 64

_HI = jax.lax.Precision.HIGHEST

_SC_PARAMS = pltpu.CompilerParams()
if "needs_layout_passes" in pltpu.CompilerParams.__dataclass_fields__:
    _SC_PARAMS = dataclasses.replace(_SC_PARAMS, needs_layout_passes=False)


def _pad_rows(n):
    q = NS * 8
    return ((n + q - 1) // q) * q


@functools.partial(jax.jit, static_argnames=("n", "e"))
def _sc_degree(dst, *, n, e):
    npad = _pad_rows(n)
    ept = e // NW

    @functools.partial(
        pl.kernel,
        out_type=jax.ShapeDtypeStruct((NW, npad), jnp.float32),
        mesh=plsc.VectorSubcoreMesh(core_axis_name="c", subcore_axis_name="s"),
        scratch_types=[
            pltpu.VMEM((ept,), jnp.int32),
            pltpu.VMEM((npad,), jnp.float32),
            pltpu.SemaphoreType.DMA,
        ],
        compiler_params=_SC_PARAMS,
    )
    def deg_kernel(dst_hbm, out_hbm, idx_v, hist, sem):
        cid = lax.axis_index("c")
        sid = lax.axis_index("s")
        wid = sid * NC + cid
        pltpu.sync_copy(dst_hbm.at[pl.ds(wid * ept, ept)], idx_v)
        zeros16 = jnp.zeros((16,), jnp.float32)

        @pl.loop(0, npad // 16)
        def _(i):
            hist[pl.ds(i * 16, 16)] = zeros16

        ones16 = jnp.ones((16,), jnp.float32)

        @pl.loop(0, ept // 16)
        def _(j):
            idx = idx_v[pl.ds(j * 16, 16)]
            plsc.addupdate_scatter(hist, [idx], ones16)

        pltpu.sync_copy(hist, out_hbm.at[wid])

    return deg_kernel(dst)


@functools.partial(jax.jit, static_argnames=("n", "nch", "h"))
def _sc_edge_agg(zp, src3, dst3, zeros_hbm, *, n, nch, h):
    npad = _pad_rows(n)
    rpt = npad // NS

    assert nch % 2 == 1
    ept = nch * CH

    @functools.partial(
        pl.kernel,
        out_type=jax.ShapeDtypeStruct((NC, npad, h), jnp.float32),
        mesh=plsc.VectorSubcoreMesh(core_axis_name="c", subcore_axis_name="s"),
        scratch_types=[
            pltpu.VMEM((ept,), jnp.int32),
            pltpu.VMEM((nch, CH), jnp.int32),
            pltpu.VMEM((2, CH, h), jnp.float32),
            pltpu.VMEM_SHARED((npad, h), jnp.float32),
            pltpu.SemaphoreType.DMA,
            pltpu.SemaphoreType.DMA,
        ],
    )
    def agg_kernel(zp_hbm, src_hbm, dst_hbm, zeros_h, out_hbm,
                   srcv, dstv, rows, acc, gs0, gs1):
        cid = lax.axis_index("c")
        sid = lax.axis_index("s")
        wid = sid * NC + cid
        r0 = sid * rpt

        def issue_gather(c, rb, sem):
            idx = srcv.at[pl.ds(c * CH, CH)]
            pltpu.async_copy(zp_hbm.at[idx], rows.at[rb], sem)

        def wait_gather(rb, sem):
            pltpu.make_async_copy(zp_hbm.at[srcv.at[pl.ds(0, CH)]],
                                  rows.at[rb], sem).wait()

        pltpu.sync_copy(src_hbm.at[wid], srcv)
        pltpu.sync_copy(dst_hbm.at[wid], dstv)
        pltpu.sync_copy(zeros_h.at[pl.ds(r0, rpt)], acc.at[pl.ds(r0, rpt)])
        plsc.subcore_barrier()

        issue_gather(0, 0, gs0)

        @pl.loop(0, nch // 2)
        def _(k):
            j = 2 * k
            issue_gather(j + 1, 1, gs1)
            wait_gather(0, gs0)
            pltpu.sync_copy(rows.at[0], acc.at[dstv.at[j]], add=True)
            issue_gather(j + 2, 0, gs0)
            wait_gather(1, gs1)
            pltpu.sync_copy(rows.at[1], acc.at[dstv.at[j + 1]], add=True)

        wait_gather(0, gs0)
        pltpu.sync_copy(rows.at[0], acc.at[dstv.at[nch - 1]], add=True)

        plsc.subcore_barrier()
        pltpu.sync_copy(acc.at[pl.ds(r0, rpt)], out_hbm.at[cid, pl.ds(r0, rpt)])

    return agg_kernel(zp, src3, dst3, zeros_hbm)


def _tc_matmul(x, w):
    def body(x_ref, w_ref, z_ref):
        z_ref[...] = jnp.dot(x_ref[...], w_ref[...], precision=_HI,
                             preferred_element_type=jnp.float32)

    return pl.pallas_call(
        body,
        out_shape=jax.ShapeDtypeStruct((x.shape[0], w.shape[1]), jnp.float32),
    )(x, w)


def _tc_prep(degp, ones_nw, z):
    n, h = z.shape

    npad = _pad_rows(n)

    def body(degp_ref, ones_ref, z_ref, dinv_ref, zp_ref):
        degc = jax.lax.dot_general(
            degp_ref[...], ones_ref[...],
            dimension_numbers=(((0,), (0,)), ((), ())),
            precision=_HI, preferred_element_type=jnp.float32)
        dinv = jax.lax.rsqrt(degc[:n] + 1.0)
        dinv_ref[...] = dinv
        zp_ref[...] = jnp.concatenate(
            [z_ref[...] * dinv, jnp.zeros((npad - n, h), jnp.float32)], axis=0)

    return pl.pallas_call(
        body,
        out_shape=(jax.ShapeDtypeStruct((n, 1), jnp.float32),
                   jax.ShapeDtypeStruct((npad, h), jnp.float32)),
    )(degp, ones_nw, z)


def _combine_bn_relu(p_ref, z_ref, dinv_ref, g_ref, be_ref):
    n = z_ref.shape[0]
    dinv = dinv_ref[...]
    pre = dinv * (p_ref[0, :n] + p_ref[1, :n]) + (dinv * dinv) * z_ref[...]
    m = jnp.mean(pre, axis=0, keepdims=True)
    v = jnp.mean((pre - m) ** 2, axis=0, keepdims=True)
    hh = g_ref[...] * (pre - m) * jax.lax.rsqrt(v + 1e-5) + be_ref[...]
    return jnp.maximum(hh, 0.0)


def _tc_mid(p, z, dinv, g, be, w_next):
    n, h = z.shape
    npad = _pad_rows(n)
    hn = w_next.shape[1]

    def body(p_ref, z_ref, dinv_ref, g_ref, be_ref, w_ref, zn_ref, zpn_ref):
        hh = _combine_bn_relu(p_ref, z_ref, dinv_ref, g_ref, be_ref)
        zn = jnp.dot(hh, w_ref[...], precision=_HI,
                     preferred_element_type=jnp.float32)
        zn_ref[...] = zn
        zpn_ref[...] = jnp.concatenate(
            [zn * dinv_ref[...], jnp.zeros((npad - n, hn), jnp.float32)],
            axis=0)

    return pl.pallas_call(
        body,
        out_shape=(jax.ShapeDtypeStruct((n, hn), jnp.float32),
                   jax.ShapeDtypeStruct((npad, hn), jnp.float32)),
    )(p, z, dinv, g, be, w_next)


def _tc_final(p, z, dinv, g, be, batch2d, wl, bl):
    n, h = z.shape

    def body(p_ref, z_ref, dinv_ref, g_ref, be_ref, b_ref, wl_ref, bl_ref,
             out_ref):
        hh = _combine_bn_relu(p_ref, z_ref, dinv_ref, g_ref, be_ref)
        gids = jax.lax.broadcasted_iota(jnp.int32, (G, n), 0)
        onehot = jnp.where(gids == b_ref[...], 1.0, 0.0)
        sums = jnp.dot(onehot, hh, precision=_HI,
                       preferred_element_type=jnp.float32)
        counts = jnp.sum(onehot, axis=1, keepdims=True)
        pooled = sums / jnp.maximum(counts, 1.0)
        out_ref[...] = jnp.dot(pooled, wl_ref[...], precision=_HI,
                               preferred_element_type=jnp.float32) + bl_ref[...]

    return pl.pallas_call(
        body,
        out_shape=jax.ShapeDtypeStruct((G, 1), jnp.float32),
    )(p, z, dinv, g, be, batch2d, wl, bl)


def kernel(x, edge_index, batch, W1, b1, g1, be1, W2, b2, g2, be2,
           W3, b3, g3, be3, Wl, bl):
    n, f_in = x.shape
    e = edge_index.shape[1]
    h = W1.shape[1]
    npad = _pad_rows(n)

    assert e % NW == 0
    ept = e // NW
    grain = GRP * CH
    eptp = ((ept + grain - 1) // grain) * grain
    nch = eptp // CH
    dpt = eptp - ept
    assert dpt == 0 or npad > n
    idt = edge_index.dtype
    dummy_src = jnp.full((NW, dpt), n, dtype=idt)
    dummy_dst = jnp.arange(NW * dpt, dtype=idt).reshape(NW, dpt) % npad
    src3 = jnp.concatenate(
        [edge_index[0].reshape(NW, ept), dummy_src], axis=1)
    dst3 = jnp.concatenate(
        [edge_index[1].reshape(NW, ept), dummy_dst], axis=1).reshape(NW, nch, CH)
    zeros_h = jnp.zeros((npad, h), jnp.float32)
    ones_nw = jnp.ones((NW, 1), jnp.float32)
    batch2d = batch.reshape(1, n)
    g1r, be1r = g1.reshape(1, h), be1.reshape(1, h)
    g2r, be2r = g2.reshape(1, h), be2.reshape(1, h)
    g3r, be3r = g3.reshape(1, h), be3.reshape(1, h)
    blr = bl.reshape(1, 1)

    degp = _sc_degree(edge_index[1], n=n, e=e)
    z1 = _tc_matmul(x, W1)
    dinv, zp1 = _tc_prep(degp, ones_nw, z1)
    p1 = _sc_edge_agg(zp1, src3, dst3, zeros_h, n=n, nch=nch, h=h)
    z2, zp2 = _tc_mid(p1, z1, dinv, g1r, be1r, W2)
    p2 = _sc_edge_agg(zp2, src3, dst3, zeros_h, n=n, nch=nch, h=h)
    z3, zp3 = _tc_mid(p2, z2, dinv, g2r, be2r, W3)
    p3 = _sc_edge_agg(zp3, src3, dst3, zeros_h, n=n, nch=nch, h=h)
    return _tc_final(p3, z3, dinv, g3r, be3r, batch2d, Wl, blr)
